# baseline (speedup 1.0000x reference)
import functools

import jax
import jax.numpy as jnp
from jax.experimental import pallas as pl

K = 16
KK = 32


def _knn_body(x_ref, out_ref):
    xb = x_ref[0]
    C, N = xb.shape
    norm = jnp.sqrt(jnp.sum(xb * xb, axis=0, keepdims=True))
    xn = xb / jnp.maximum(norm, 1e-12)
    inner = jax.lax.dot_general(
        xn, xn,
        dimension_numbers=(((0,), (0,)), ((), ())),
        preferred_element_type=jnp.float32,
    )
    x_inner = -2.0 * inner
    sq = jnp.sum(xn * xn, axis=0, keepdims=True)
    dist = (jnp.transpose(sq) + x_inner) + sq
    score = -dist
    col = jax.lax.broadcasted_iota(jnp.int32, (N, N), 1)
    neg_inf = jnp.float32(-jnp.inf)
    m = jnp.max(score, axis=1, keepdims=True)
    prev_idx = jnp.full((N, 1), -1, jnp.int32)
    cols_out = []
    for k in range(KK):
        eq = score == m
        elig = eq & (col > prev_idx)
        idx = jnp.min(jnp.where(elig, col, N), axis=1, keepdims=True)
        if k % 2 == 0:
            cols_out.append(idx)
        if k != KK - 1:
            maxcol = jnp.max(jnp.where(eq, col, -1), axis=1, keepdims=True)
            mstrict = jnp.max(jnp.where(score < m, score, neg_inf),
                              axis=1, keepdims=True)
            has_tie = maxcol > idx
            prev_idx = jnp.where(has_tie, idx, -1)
            m = jnp.where(has_tie, m, mstrict)
    out_ref[0] = jnp.concatenate(cols_out, axis=1)


@jax.jit
def kernel(x):
    B, C, N, _ = x.shape
    xs = jnp.squeeze(x, -1)
    nn_idx = pl.pallas_call(
        _knn_body,
        grid=(B,),
        in_specs=[pl.BlockSpec((1, C, N), lambda b: (b, 0, 0))],
        out_specs=pl.BlockSpec((1, N, K), lambda b: (b, 0, 0)),
        out_shape=jax.ShapeDtypeStruct((B, N, K), jnp.int32),
    )(xs)
    center_idx = jnp.broadcast_to(
        jnp.arange(N, dtype=jnp.int32)[None, :, None], (B, N, K)
    )
    return jnp.stack((nn_idx, center_idx), axis=0)


# speedup vs baseline: 1.2946x; 1.2946x over previous
import functools

import jax
import jax.numpy as jnp
from jax.experimental import pallas as pl

K = 16
KK = 32


def _knn_body(x_ref, out_ref):
    xb = x_ref[0]
    C, N = xb.shape
    norm = jnp.sqrt(jnp.sum(xb * xb, axis=0, keepdims=True))
    xn = xb / jnp.maximum(norm, 1e-12)
    inner = jax.lax.dot_general(
        xn, xn,
        dimension_numbers=(((0,), (0,)), ((), ())),
        preferred_element_type=jnp.float32,
    )
    x_inner = -2.0 * inner
    sq = jnp.sum(xn * xn, axis=0, keepdims=True)
    dist = (jnp.transpose(sq) + x_inner) + sq
    score = -dist
    col = jax.lax.broadcasted_iota(jnp.int32, (N, N), 1)
    neg_inf = jnp.float32(-jnp.inf)
    cols_out = []
    m1 = jnp.max(score, axis=1, keepdims=True)
    for t in range(K):
        idx1 = jnp.min(jnp.where(score == m1, col, N), axis=1, keepdims=True)
        cols_out.append(idx1)
        hide1 = col == idx1
        m2 = jnp.max(jnp.where(hide1, neg_inf, score), axis=1, keepdims=True)
        idx2 = jnp.min(
            jnp.where((score == m2) & ~hide1, col, N), axis=1, keepdims=True
        )
        if t != K - 1:
            score = jnp.where(hide1 | (col == idx2), neg_inf, score)
            m1 = jnp.max(score, axis=1, keepdims=True)
    out_ref[0] = jnp.concatenate(cols_out, axis=1)


@jax.jit
def kernel(x):
    B, C, N, _ = x.shape
    xs = jnp.squeeze(x, -1)
    nn_idx = pl.pallas_call(
        _knn_body,
        grid=(B,),
        in_specs=[pl.BlockSpec((1, C, N), lambda b: (b, 0, 0))],
        out_specs=pl.BlockSpec((1, N, K), lambda b: (b, 0, 0)),
        out_shape=jax.ShapeDtypeStruct((B, N, K), jnp.int32),
    )(xs)
    center_idx = jnp.broadcast_to(
        jnp.arange(N, dtype=jnp.int32)[None, :, None], (B, N, K)
    )
    return jnp.stack((nn_idx, center_idx), axis=0)


# speedup vs baseline: 2.4698x; 1.9078x over previous
import functools

import jax
import jax.numpy as jnp
from jax.experimental import pallas as pl

K = 16
KK = 32


def _knn_body(x_ref, out_ref):
    xb = x_ref[0]
    C, N = xb.shape
    norm = jnp.sqrt(jnp.sum(xb * xb, axis=0, keepdims=True))
    xn = xb / jnp.maximum(norm, 1e-12)
    inner = jax.lax.dot_general(
        xn, xn,
        dimension_numbers=(((0,), (0,)), ((), ())),
        preferred_element_type=jnp.float32,
    )
    x_inner = -2.0 * inner
    sq = jnp.sum(xn * xn, axis=0, keepdims=True)
    dist = (jnp.transpose(sq) + x_inner) + sq
    score = -dist
    col = jax.lax.broadcasted_iota(jnp.int32, (N, N), 1)
    neg_inf = jnp.float32(-jnp.inf)
    cols_out = []
    for k in range(KK):
        idx = jnp.argmax(score, axis=1, keepdims=True).astype(jnp.int32)
        if k % 2 == 0:
            cols_out.append(idx)
        if k != KK - 1:
            score = jnp.where(col == idx, neg_inf, score)
    out_ref[0] = jnp.concatenate(cols_out, axis=1)


@jax.jit
def kernel(x):
    B, C, N, _ = x.shape
    xs = jnp.squeeze(x, -1)
    nn_idx = pl.pallas_call(
        _knn_body,
        grid=(B,),
        in_specs=[pl.BlockSpec((1, C, N), lambda b: (b, 0, 0))],
        out_specs=pl.BlockSpec((1, N, K), lambda b: (b, 0, 0)),
        out_shape=jax.ShapeDtypeStruct((B, N, K), jnp.int32),
    )(xs)
    center_idx = jnp.broadcast_to(
        jnp.arange(N, dtype=jnp.int32)[None, :, None], (B, N, K)
    )
    return jnp.stack((nn_idx, center_idx), axis=0)
